# baseline (speedup 1.0000x reference)
import functools

import jax
import jax.numpy as jnp
from jax.experimental import pallas as pl
from jax.experimental.pallas import tpu as pltpu

B, V = 4096, 100
NUM_NAMES, NUM_ROLES, NUM_GROUPS = 100, 8, 8
NAME_D, ROLE_D, GROUP_D = 32, 16, 16
HID, TOK = 128, 64
ROWS = B * V
BLOCK_B = 128
BLOCK_ROWS = BLOCK_B * V
EPS = 1e-5


def _encoder_kernel(vals_ref, nidx_ref, ridx_ref, gidx_ref,
                    ntabT_ref, rtabT_ref, gtabT_ref,
                    w1row_ref, w1colg_ref, w1nT_ref, w1rT_ref, w1gT_ref,
                    b1c_ref, g1c_ref, be1c_ref,
                    w2T_ref, b2c_ref, g2c_ref, be2c_ref,
                    w3_ref, b3r_ref,
                    out_ref, baseT_s, mb_s, cv2_s, vbe_s):
    dot = functools.partial(jax.lax.dot, preferred_element_type=jnp.float32)
    ones_row = jnp.full((1, HID), 1.0 / HID, dtype=jnp.float32)

    @pl.when(pl.program_id(0) == 0)
    def _prologue():
        ion = jax.lax.broadcasted_iota(jnp.int32, (NUM_NAMES, V), 0)
        ior = jax.lax.broadcasted_iota(jnp.int32, (NUM_ROLES, V), 0)
        iog = jax.lax.broadcasted_iota(jnp.int32, (NUM_GROUPS, V), 0)
        ohnT = (nidx_ref[...] == ion).astype(jnp.float32)
        ohrT = (ridx_ref[...] == ior).astype(jnp.float32)
        ohgT = (gidx_ref[...] == iog).astype(jnp.float32)
        baseT = (dot(w1nT_ref[...], dot(ntabT_ref[...], ohnT))
                 + dot(w1rT_ref[...], dot(rtabT_ref[...], ohrT))
                 + dot(w1gT_ref[...], dot(gtabT_ref[...], ohgT))
                 + b1c_ref[...])
        w1r = w1row_ref[...]
        mw = jnp.mean(w1r, axis=1, keepdims=True)
        mbv = dot(ones_row, baseT)
        cvv = dot(ones_row * w1r, baseT) - mw * mbv
        vbv = dot(ones_row, baseT * baseT) - mbv * mbv + EPS
        ior2 = jax.lax.broadcasted_iota(jnp.int32, (V, BLOCK_ROWS), 0)
        ioc2 = jax.lax.broadcasted_iota(jnp.int32, (V, BLOCK_ROWS), 1)
        selT = (jax.lax.rem(ioc2, V) == ior2).astype(jnp.float32)
        baseT_s[...] = dot(baseT * g1c_ref[...], selT)
        mb_s[...] = dot(mbv, selT)
        cv2_s[...] = dot(2.0 * cvv, selT)
        vbe_s[...] = dot(vbv, selT)

    w1r = w1row_ref[...]
    mw = jnp.mean(w1r, axis=1, keepdims=True)
    vw = jnp.mean(w1r * w1r, axis=1, keepdims=True) - mw * mw

    vrow = vals_ref[...]
    m1 = vrow * mw + mb_s[...]
    var1 = (vrow * vw + cv2_s[...]) * vrow + vbe_s[...]
    inv1 = jax.lax.rsqrt(var1)
    mm = m1 * inv1

    h = dot(w1colg_ref[...], vrow) + baseT_s[...]
    h = jnp.maximum(h * inv1 - dot(g1c_ref[...], mm) + be1c_ref[...], 0.0)
    h = dot(w2T_ref[...], h.astype(jnp.bfloat16)) + b2c_ref[...]

    m2 = dot(ones_row, h)
    q2 = dot(ones_row, h * h)
    inv2 = jax.lax.rsqrt(q2 - m2 * m2 + EPS)
    mm2 = m2 * inv2
    h = jnp.maximum((h * inv2 - mm2) * g2c_ref[...] + be2c_ref[...], 0.0)
    o = jax.lax.dot_general(h.astype(jnp.bfloat16), w3_ref[...],
                            (((0,), (0,)), ((), ())),
                            preferred_element_type=jnp.float32)
    out_ref[...] = (o + b3r_ref[...]).reshape(BLOCK_B, V, TOK)


def kernel(values, name_idx, role_idx, group_idx, name_table, role_table,
           group_table, W1, b1, g1, be1, W2, b2, g2, be2, W3, b3):
    vals = values.reshape(1, ROWS)
    grid = B // BLOCK_B

    row_spec = pl.BlockSpec((1, BLOCK_ROWS), lambda i: (0, i))
    out_spec = pl.BlockSpec((BLOCK_B, V, TOK), lambda i: (i, 0, 0))

    def full(shape):
        return pl.BlockSpec(shape, lambda i: (0,) * len(shape))

    out = pl.pallas_call(
        _encoder_kernel,
        grid=(grid,),
        in_specs=[
            row_spec,
            full((1, V)), full((1, V)), full((1, V)),
            full((NAME_D, NUM_NAMES)), full((ROLE_D, NUM_ROLES)),
            full((GROUP_D, NUM_GROUPS)),
            full((1, HID)), full((HID, 1)), full((HID, NAME_D)),
            full((HID, ROLE_D)), full((HID, GROUP_D)),
            full((HID, 1)), full((HID, 1)), full((HID, 1)),
            full((HID, HID)), full((HID, 1)), full((HID, 1)), full((HID, 1)),
            full((HID, TOK)), full((1, TOK)),
        ],
        out_specs=out_spec,
        out_shape=jax.ShapeDtypeStruct((B, V, TOK), jnp.float32),
        scratch_shapes=[
            pltpu.VMEM((HID, BLOCK_ROWS), jnp.float32),
            pltpu.VMEM((1, BLOCK_ROWS), jnp.float32),
            pltpu.VMEM((1, BLOCK_ROWS), jnp.float32),
            pltpu.VMEM((1, BLOCK_ROWS), jnp.float32),
        ],
        compiler_params=pltpu.CompilerParams(
            dimension_semantics=("arbitrary",),
        ),
    )(
        vals,
        name_idx.reshape(1, V), role_idx.reshape(1, V),
        group_idx.reshape(1, V),
        name_table.T, role_table.T, group_table.T,
        W1[0:1, :], (W1[0:1, :] * g1.reshape(1, HID)).T,
        W1[1:1 + NAME_D, :].T,
        W1[1 + NAME_D:1 + NAME_D + ROLE_D, :].T,
        W1[1 + NAME_D + ROLE_D:, :].T,
        b1.reshape(HID, 1), g1.reshape(HID, 1), be1.reshape(HID, 1),
        W2.T.astype(jnp.bfloat16), b2.reshape(HID, 1), g2.reshape(HID, 1),
        be2.reshape(HID, 1),
        W3.astype(jnp.bfloat16), b3.reshape(1, TOK),
    )
    return out
